# Initial kernel scaffold; baseline (speedup 1.0000x reference)
import jax
import jax.numpy as jnp
from jax.experimental import pallas as pl


def kernel(coords):
    raise NotImplementedError("write your pallas kernel here")


# speedup vs baseline: 7.5632x; 7.5632x over previous
import functools

import jax
import jax.numpy as jnp
from jax import lax
from jax.experimental import pallas as pl
from jax.experimental.pallas import tpu as pltpu
from jax.experimental.pallas import tpu_sc as plsc

L = 4096
N = 16
K = 1024
LANES = 16
CHUNKS = L // LANES


def _fps_body(xs_hbm, ys_hbm, zs_hbm, out_hbm, x_v, y_v, z_v, dist_v, sel_v):
    wid = lax.axis_index("s") * 2 + lax.axis_index("c")

    @pl.when(wid < N)
    def _():
        cloud = wid
        pltpu.sync_copy(xs_hbm.at[cloud], x_v)
        pltpu.sync_copy(ys_hbm.at[cloud], y_v)
        pltpu.sync_copy(zs_hbm.at[cloud], z_v)

        lane = lax.iota(jnp.int32, 16)
        lane0 = lane == 0
        inf16 = jnp.full((16,), jnp.inf, dtype=jnp.float32)

        def init_body(c, carry):
            dist_v[pl.ds(c * 16, 16)] = inf16
            return carry

        lax.fori_loop(0, CHUNKS, init_body, 0, unroll=8)

        zero16 = jnp.zeros((16,), dtype=jnp.int32)
        plsc.store_scatter(sel_v, [zero16], zero16, mask=lane0)

        def step(i, j_prev):
            jv = jnp.full((16,), j_prev, dtype=jnp.int32)
            lastx = plsc.load_gather(x_v, [jv])
            lasty = plsc.load_gather(y_v, [jv])
            lastz = plsc.load_gather(z_v, [jv])

            def chunk(c, carry):
                vmax, vidx = carry
                s = pl.ds(c * 16, 16)
                dx = x_v[s] - lastx
                dy = y_v[s] - lasty
                dz = z_v[s] - lastz
                d = (dx * dx + dy * dy) + dz * dz
                dn = jnp.minimum(dist_v[s], d)
                dist_v[s] = dn
                m = dn > vmax
                vmax = jnp.where(m, dn, vmax)
                vidx = jnp.where(m, c * 16 + lane, vidx)
                return (vmax, vidx)

            vmax, vidx = lax.fori_loop(
                0, CHUNKS, chunk,
                (jnp.full((16,), -jnp.inf, dtype=jnp.float32), zero16),
                unroll=4,
            )
            mx = jnp.max(vmax)
            cand = jnp.where(vmax == mx, vidx, jnp.int32(2**30))
            nxt = jnp.min(cand)
            plsc.store_scatter(
                sel_v,
                [jnp.full((16,), i, dtype=jnp.int32)],
                jnp.full((16,), nxt, dtype=jnp.int32),
                mask=lane0,
            )
            return nxt

        lax.fori_loop(1, K, step, jnp.int32(0))
        pltpu.sync_copy(sel_v, out_hbm.at[cloud])


@jax.jit
def _fps_all(xs, ys, zs):
    mesh = plsc.VectorSubcoreMesh(core_axis_name="c", subcore_axis_name="s")
    return pl.kernel(
        _fps_body,
        out_type=jax.ShapeDtypeStruct((N, K), jnp.int32),
        mesh=mesh,
        compiler_params=pltpu.CompilerParams(needs_layout_passes=False),
        scratch_types=[
            pltpu.VMEM((L,), jnp.float32),
            pltpu.VMEM((L,), jnp.float32),
            pltpu.VMEM((L,), jnp.float32),
            pltpu.VMEM((L,), jnp.float32),
            pltpu.VMEM((K,), jnp.int32),
        ],
    )(xs, ys, zs)


def kernel(coords):
    c = jax.lax.stop_gradient(coords)
    pts = jnp.transpose(c, (1, 2, 0))
    xs = pts[:, 0, :]
    ys = pts[:, 1, :]
    zs = pts[:, 2, :]
    keep = _fps_all(xs, ys, zs)
    keep = jnp.transpose(keep, (1, 0))
    gk, gn = jnp.meshgrid(jnp.arange(K), jnp.arange(N), indexing="ij")
    return (
        keep.reshape(-1).astype(jnp.int64),
        gn.reshape(-1).astype(jnp.int64),
    )


# speedup vs baseline: 19.9381x; 2.6362x over previous
import functools

import jax
import jax.numpy as jnp
from jax import lax
from jax.experimental import pallas as pl
from jax.experimental.pallas import tpu as pltpu
from jax.experimental.pallas import tpu_sc as plsc

L = 4096
N = 16
K = 1024
LANES = 16
CHUNKS = L // LANES


def _fps_body(xs_hbm, ys_hbm, zs_hbm, out_hbm, x_v, y_v, z_v, dist_v, sel_v):
    wid = lax.axis_index("s") * 2 + lax.axis_index("c")

    @pl.when(wid < N)
    def _():
        cloud = wid
        pltpu.sync_copy(xs_hbm.at[cloud], x_v)
        pltpu.sync_copy(ys_hbm.at[cloud], y_v)
        pltpu.sync_copy(zs_hbm.at[cloud], z_v)

        lane = lax.iota(jnp.int32, 16)
        lane0 = lane == 0
        inf16 = jnp.full((16,), jnp.inf, dtype=jnp.float32)

        @plsc.parallel_loop(0, CHUNKS, unroll=8)
        def init_body(c):
            dist_v[pl.ds(c * 16, 16)] = inf16

        zero16 = jnp.zeros((16,), dtype=jnp.int32)
        plsc.store_scatter(sel_v, [zero16], zero16, mask=lane0)

        def step(i, j_prev):
            jv = jnp.full((16,), j_prev, dtype=jnp.int32)
            lastx = plsc.load_gather(x_v, [jv])
            lasty = plsc.load_gather(y_v, [jv])
            lastz = plsc.load_gather(z_v, [jv])

            init_carry = (
                jnp.full((16,), -jnp.inf, dtype=jnp.float32),
                jnp.full((16,), jnp.int32(2**30), dtype=jnp.int32),
            )

            @plsc.parallel_loop(0, CHUNKS, unroll=8, carry=init_carry)
            def chunk(c, carry):
                vmax, vidx = carry
                s = pl.ds(c * 16, 16)
                dx = x_v[s] - lastx
                dy = y_v[s] - lasty
                dz = z_v[s] - lastz
                d = (dx * dx + dy * dy) + dz * dz
                dn = jnp.minimum(dist_v[s], d)
                dist_v[s] = dn
                cidx = c * 16 + lane
                m = (dn > vmax) | ((dn == vmax) & (cidx < vidx))
                vmax = jnp.where(m, dn, vmax)
                vidx = jnp.where(m, cidx, vidx)
                return (vmax, vidx)

            vmax, vidx = chunk
            mx = jnp.max(vmax)
            cand = jnp.where(vmax == mx, vidx, jnp.int32(2**30))
            nxt = jnp.min(cand)
            plsc.store_scatter(
                sel_v,
                [jnp.full((16,), i, dtype=jnp.int32)],
                jnp.full((16,), nxt, dtype=jnp.int32),
                mask=lane0,
            )
            return nxt

        lax.fori_loop(1, K, step, jnp.int32(0))
        pltpu.sync_copy(sel_v, out_hbm.at[cloud])


@jax.jit
def _fps_all(xs, ys, zs):
    mesh = plsc.VectorSubcoreMesh(core_axis_name="c", subcore_axis_name="s")
    return pl.kernel(
        _fps_body,
        out_type=jax.ShapeDtypeStruct((N, K), jnp.int32),
        mesh=mesh,
        compiler_params=pltpu.CompilerParams(needs_layout_passes=False),
        scratch_types=[
            pltpu.VMEM((L,), jnp.float32),
            pltpu.VMEM((L,), jnp.float32),
            pltpu.VMEM((L,), jnp.float32),
            pltpu.VMEM((L,), jnp.float32),
            pltpu.VMEM((K,), jnp.int32),
        ],
    )(xs, ys, zs)


def kernel(coords):
    c = jax.lax.stop_gradient(coords)
    pts = jnp.transpose(c, (1, 2, 0))
    xs = pts[:, 0, :]
    ys = pts[:, 1, :]
    zs = pts[:, 2, :]
    keep = _fps_all(xs, ys, zs)
    keep = jnp.transpose(keep, (1, 0))
    gk, gn = jnp.meshgrid(jnp.arange(K), jnp.arange(N), indexing="ij")
    return (
        keep.reshape(-1).astype(jnp.int64),
        gn.reshape(-1).astype(jnp.int64),
    )


# speedup vs baseline: 25.3789x; 1.2729x over previous
import functools

import jax
import jax.numpy as jnp
from jax import lax
from jax.experimental import pallas as pl
from jax.experimental.pallas import tpu as pltpu
from jax.experimental.pallas import tpu_sc as plsc

L = 4096
N = 16
K = 1024
LANES = 16
CHUNKS = L // LANES


def _fps_body(xs_hbm, ys_hbm, zs_hbm, out_hbm, x_v, y_v, z_v, dist_v, sel_v):
    wid = lax.axis_index("s") * 2 + lax.axis_index("c")

    @pl.when(wid < N)
    def _():
        cloud = wid
        pltpu.sync_copy(xs_hbm.at[cloud], x_v)
        pltpu.sync_copy(ys_hbm.at[cloud], y_v)
        pltpu.sync_copy(zs_hbm.at[cloud], z_v)

        lane = lax.iota(jnp.int32, 16)
        lane0 = lane == 0
        inf16 = jnp.full((16,), jnp.inf, dtype=jnp.float32)

        @plsc.parallel_loop(0, CHUNKS, unroll=8)
        def init_body(c):
            dist_v[pl.ds(c * 16, 16)] = inf16

        zero16 = jnp.zeros((16,), dtype=jnp.int32)
        plsc.store_scatter(sel_v, [zero16], zero16, mask=lane0)

        NACC = 4
        GROUPS = CHUNKS // NACC
        neg_inf16 = jnp.full((16,), -jnp.inf, dtype=jnp.float32)

        def step(i, j_prev):
            jv = jnp.full((16,), j_prev, dtype=jnp.int32)
            lastx = plsc.load_gather(x_v, [jv])
            lasty = plsc.load_gather(y_v, [jv])
            lastz = plsc.load_gather(z_v, [jv])

            init_carry = ((neg_inf16,) * NACC, (zero16,) * NACC)

            @plsc.parallel_loop(0, GROUPS, unroll=4, carry=init_carry)
            def chunk(g, carry):
                vmaxs, vcbs = carry
                vmaxs, vcbs = list(vmaxs), list(vcbs)
                gvec = jnp.full((16,), g, dtype=jnp.int32)
                for j in range(NACC):
                    s = pl.ds((g * NACC + j) * 16, 16)
                    dx = x_v[s] - lastx
                    dy = y_v[s] - lasty
                    dz = z_v[s] - lastz
                    d = (dx * dx + dy * dy) + dz * dz
                    dn = jnp.minimum(dist_v[s], d)
                    dist_v[s] = dn
                    m = dn > vmaxs[j]
                    vmaxs[j] = jnp.where(m, dn, vmaxs[j])
                    vcbs[j] = jnp.where(m, gvec, vcbs[j])
                return (tuple(vmaxs), tuple(vcbs))

            vmaxs, vcbs = chunk
            pairs = [
                (vmaxs[j], vcbs[j] * (NACC * 16) + (j * 16) + lane)
                for j in range(NACC)
            ]

            def merge(a, b):
                av, ai = a
                bv, bi = b
                m = (av > bv) | ((av == bv) & (ai < bi))
                return (jnp.where(m, av, bv), jnp.where(m, ai, bi))

            vmax, vidx = merge(merge(pairs[0], pairs[1]),
                               merge(pairs[2], pairs[3]))
            mx = jnp.max(vmax)
            cand = jnp.where(vmax == mx, vidx, jnp.int32(2**30))
            nxt = jnp.min(cand)
            plsc.store_scatter(
                sel_v,
                [jnp.full((16,), i, dtype=jnp.int32)],
                jnp.full((16,), nxt, dtype=jnp.int32),
                mask=lane0,
            )
            return nxt

        lax.fori_loop(1, K, step, jnp.int32(0))
        pltpu.sync_copy(sel_v, out_hbm.at[cloud])


@jax.jit
def _fps_all(xs, ys, zs):
    mesh = plsc.VectorSubcoreMesh(core_axis_name="c", subcore_axis_name="s")
    return pl.kernel(
        _fps_body,
        out_type=jax.ShapeDtypeStruct((N, K), jnp.int32),
        mesh=mesh,
        compiler_params=pltpu.CompilerParams(needs_layout_passes=False),
        scratch_types=[
            pltpu.VMEM((L,), jnp.float32),
            pltpu.VMEM((L,), jnp.float32),
            pltpu.VMEM((L,), jnp.float32),
            pltpu.VMEM((L,), jnp.float32),
            pltpu.VMEM((K,), jnp.int32),
        ],
    )(xs, ys, zs)


def kernel(coords):
    c = jax.lax.stop_gradient(coords)
    pts = jnp.transpose(c, (1, 2, 0))
    xs = pts[:, 0, :]
    ys = pts[:, 1, :]
    zs = pts[:, 2, :]
    keep = _fps_all(xs, ys, zs)
    keep = jnp.transpose(keep, (1, 0))
    gk, gn = jnp.meshgrid(jnp.arange(K), jnp.arange(N), indexing="ij")
    return (
        keep.reshape(-1).astype(jnp.int64),
        gn.reshape(-1).astype(jnp.int64),
    )


# speedup vs baseline: 38.6893x; 1.5245x over previous
import jax
import jax.numpy as jnp
from jax import lax
from jax.experimental import pallas as pl
from jax.experimental.pallas import tpu as pltpu
from jax.experimental.pallas import tpu_sc as plsc

L = 4096
N = 16
K = 1024
LANES = 16
HALF = L // 2
HCHUNKS = HALF // LANES
NACC = 4
GROUPS = HCHUNKS // NACC
BIG = 2**30


def _fps_body(xs_hbm, ys_hbm, zs_hbm, out_hbm,
              x_v, y_v, z_v, dist_v, sel_v, xch_sm):
    c = lax.axis_index("c")
    s = lax.axis_index("s")
    cloud = c * 8 + lax.shift_right_logical(s, 1)
    h = jnp.bitwise_and(s, 1)

    pltpu.sync_copy(xs_hbm.at[cloud], x_v)
    pltpu.sync_copy(ys_hbm.at[cloud], y_v)
    pltpu.sync_copy(zs_hbm.at[cloud], z_v)

    lane = lax.iota(jnp.int32, 16)
    lane0 = lane == 0
    zero16 = jnp.zeros((16,), dtype=jnp.int32)
    one16 = jnp.full((16,), 1, dtype=jnp.int32)
    inf16 = jnp.full((16,), jnp.inf, dtype=jnp.float32)
    neg_inf16 = jnp.full((16,), -jnp.inf, dtype=jnp.float32)
    base = h * HALF
    lane_off = lane + base

    @plsc.parallel_loop(0, HCHUNKS, unroll=8)
    def init_body(ch):
        dist_v[pl.ds(ch * 16, 16)] = inf16

    @pl.when(h == 0)
    def _():
        plsc.store_scatter(sel_v, [zero16], zero16, mask=lane0)

    xch_sm[0] = jnp.int32(0)
    xch_sm[1] = jnp.int32(0)
    xch_sm[2] = jnp.int32(0)
    xch_sm[3] = jnp.int32(0)
    plsc.subcore_barrier()
    partner = jnp.bitwise_xor(s, 1)

    def step(i, j_prev):
        jv = jnp.full((16,), j_prev, dtype=jnp.int32)
        lastx = plsc.load_gather(x_v, [jv])
        lasty = plsc.load_gather(y_v, [jv])
        lastz = plsc.load_gather(z_v, [jv])

        init_carry = ((neg_inf16,) * NACC, (zero16,) * NACC)

        @plsc.parallel_loop(0, GROUPS, unroll=4, carry=init_carry)
        def chunk(g, carry):
            vmaxs, vcbs = carry
            vmaxs, vcbs = list(vmaxs), list(vcbs)
            gvec = jnp.full((16,), g, dtype=jnp.int32)
            for j in range(NACC):
                off = (g * NACC + j) * 16
                sl = pl.ds(off, 16)
                sg = pl.ds(base + off, 16)
                dx = x_v[sg] - lastx
                dy = y_v[sg] - lasty
                dz = z_v[sg] - lastz
                d = (dx * dx + dy * dy) + dz * dz
                dn = jnp.minimum(dist_v[sl], d)
                dist_v[sl] = dn
                m = dn > vmaxs[j]
                vmaxs[j] = jnp.where(m, dn, vmaxs[j])
                vcbs[j] = jnp.where(m, gvec, vcbs[j])
            return (tuple(vmaxs), tuple(vcbs))

        vmaxs, vcbs = chunk
        pairs = [
            (vmaxs[j], vcbs[j] * (NACC * 16) + (j * 16) + lane_off)
            for j in range(NACC)
        ]

        def merge(a, b):
            av, ai = a
            bv, bi = b
            m = (av > bv) | ((av == bv) & (ai < bi))
            return (jnp.where(m, av, bv), jnp.where(m, ai, bi))

        vmax, vidx = merge(merge(pairs[0], pairs[1]),
                           merge(pairs[2], pairs[3]))
        mx = jnp.max(vmax)
        cand = jnp.where(vmax == mx, vidx, jnp.int32(BIG))
        nxt = jnp.min(cand)

        mb = lax.bitcast_convert_type(mx, jnp.int32)
        par2 = jnp.bitwise_and(i, 1) * 2
        plsc.fetch_and_add(xch_sm.at[par2], mb, subcore_id=partner)
        plsc.fetch_and_add(xch_sm.at[par2 + 1], nxt, subcore_id=partner)
        plsc.subcore_barrier()
        ob = xch_sm[par2]
        oi = xch_sm[par2 + 1]
        xch_sm[par2] = jnp.int32(0)
        xch_sm[par2 + 1] = jnp.int32(0)
        m = (ob > mb) | ((ob == mb) & (oi < nxt))
        win = jnp.where(m, oi, nxt)

        @pl.when(h == 0)
        def _():
            plsc.store_scatter(
                sel_v, [jnp.full((16,), i, dtype=jnp.int32)],
                jnp.full((16,), win, dtype=jnp.int32),
                mask=lane0,
            )

        return win

    lax.fori_loop(1, K, step, jnp.int32(0))

    @pl.when(h == 0)
    def _():
        pltpu.sync_copy(sel_v, out_hbm.at[cloud])


@jax.jit
def _fps_all(xs, ys, zs):
    mesh = plsc.VectorSubcoreMesh(core_axis_name="c", subcore_axis_name="s")
    return pl.kernel(
        _fps_body,
        out_type=jax.ShapeDtypeStruct((N, K), jnp.int32),
        mesh=mesh,
        compiler_params=pltpu.CompilerParams(needs_layout_passes=False),
        scratch_types=[
            pltpu.VMEM((L,), jnp.float32),
            pltpu.VMEM((L,), jnp.float32),
            pltpu.VMEM((L,), jnp.float32),
            pltpu.VMEM((HALF,), jnp.float32),
            pltpu.VMEM((K,), jnp.int32),
            pltpu.SMEM((4,), jnp.int32),
        ],
    )(xs, ys, zs)


def kernel(coords):
    c = jax.lax.stop_gradient(coords)
    pts = jnp.transpose(c, (1, 2, 0))
    xs = pts[:, 0, :]
    ys = pts[:, 1, :]
    zs = pts[:, 2, :]
    keep = _fps_all(xs, ys, zs)
    keep = jnp.transpose(keep, (1, 0))
    gk, gn = jnp.meshgrid(jnp.arange(K), jnp.arange(N), indexing="ij")
    return (
        keep.reshape(-1).astype(jnp.int64),
        gn.reshape(-1).astype(jnp.int64),
    )
